# baseline (speedup 1.0000x reference)
import functools

import jax
import jax.numpy as jnp
from jax import lax
from jax.experimental import pallas as pl
from jax.experimental.pallas import tpu as pltpu
from jax.experimental.pallas import tpu_sc as plsc

_B, _C, _H, _W, _K = 8, 64, 64, 64, 1024
_HW = _H * _W
_T = 4096
_NB = _HW // _T
_N = _B * _HW
_NW = 32
_BPW = _N // _NW
_CH = 128


def _vq_block(x_ref, w_ref, idx_ref, loss_ref, a_ref, b2_ref):
    b = pl.program_id(0)
    nb = pl.program_id(1)

    @pl.when(jnp.logical_and(b == 0, nb == 0))
    def _build_consts():
        w = w_ref[...]
        ones = jnp.ones((_K, 1), jnp.float32)
        kio = jax.lax.broadcasted_iota(jnp.int32, (_K, 1), 0)
        khi = (kio // 128).astype(jnp.float32)
        klo = (kio % 128).astype(jnp.float32)
        a = jnp.concatenate([ones, khi, klo], axis=1)
        a_ref[...] = a.astype(jnp.bfloat16)
        b2_ref[...] = jnp.sum(w * w, axis=1, keepdims=True)
        loss_ref[...] = jnp.zeros_like(loss_ref)

    x = x_ref[0]
    w2 = w_ref[...] + w_ref[...]
    s2 = jnp.dot(w2, x, preferred_element_type=jnp.float32)
    a2 = jnp.sum(x * x, axis=0, keepdims=True)
    m = (a2 + b2_ref[...]) - s2
    minm = jnp.min(m, axis=0, keepdims=True)
    mask = m == minm
    oh = mask.astype(jnp.bfloat16)
    dn = (((0,), (0,)), ((), ()))
    g3 = jax.lax.dot_general(a_ref[...], oh, dn,
                             preferred_element_type=jnp.float32)
    cnt = g3[0:1, :]
    idx_ref[0, 0] = (g3[1:2, :] * 128.0 + g3[2:3, :]).astype(jnp.int32)

    @pl.when(jnp.sum(cnt) > _T + 0.5)
    def _resolve_ties():
        kiota = jax.lax.broadcasted_iota(jnp.int32, m.shape, 0).astype(
            jnp.float32)
        idxf = jnp.min(jnp.where(mask, kiota, float(_K)), axis=0,
                       keepdims=True)
        idx_ref[0, 0] = idxf.astype(jnp.int32)

    loss_ref[...] += jnp.sum(minm, axis=(0, 1), keepdims=True)


def _tc_argmin(x, weight):
    idx, loss = pl.pallas_call(
        _vq_block,
        grid=(_B, _NB),
        in_specs=[
            pl.BlockSpec((1, _C, _T), lambda b, n: (b, 0, n)),
            pl.BlockSpec((_K, _C), lambda b, n: (0, 0)),
        ],
        out_specs=[
            pl.BlockSpec((1, 1, 1, _T), lambda b, n: (b, n, 0, 0)),
            pl.BlockSpec((1, 1), lambda b, n: (0, 0)),
        ],
        out_shape=[
            jax.ShapeDtypeStruct((_B, _NB, 1, _T), jnp.int32),
            jax.ShapeDtypeStruct((1, 1), jnp.float32),
        ],
        scratch_shapes=[
            pltpu.VMEM((_K, 3), jnp.bfloat16),
            pltpu.VMEM((_K, 1), jnp.float32),
        ],
    )(x, weight)
    return idx, loss


def _sc_gather_body(idx_hbm, w_hbm, out_hbm, idx_v, rows_v, sem):
    hpw = _BPW // 2
    wid = lax.axis_index("s") * 2 + lax.axis_index("c")
    base = wid * hpw
    pltpu.sync_copy(idx_hbm.at[pl.ds(base, hpw)], idx_v)
    for j in range(hpw // _CH):
        pltpu.async_copy(
            w_hbm.at[idx_v.at[pl.ds(j * _CH, _CH)]],
            rows_v.at[pl.ds(j * _CH, _CH)], sem).wait()
    pltpu.sync_copy(rows_v, out_hbm.at[pl.ds(base, hpw)])


@functools.lru_cache(maxsize=1)
def _sc_gather():
    mesh = plsc.VectorSubcoreMesh(core_axis_name="c", subcore_axis_name="s")
    return pl.kernel(
        _sc_gather_body,
        mesh=mesh,
        out_type=jax.ShapeDtypeStruct((_N // 2, 128), jnp.float32),
        scratch_types=[
            pltpu.VMEM((_BPW // 2,), jnp.int32),
            pltpu.VMEM((_BPW // 2, 128), jnp.float32),
            pltpu.SemaphoreType.DMA,
        ],
    )


def kernel(quant_input, weight):
    x = quant_input.reshape(_B, _C, _HW)
    idx, loss = _tc_argmin(x, weight)
    idx_flat = idx.reshape(_N)
    w_pad = jnp.pad(weight, ((0, 0), (0, 128 - _C)))
    g = _sc_gather()
    rows = jnp.concatenate(
        [g(idx_flat[:_N // 2], w_pad), g(idx_flat[_N // 2:], w_pad)],
        axis=0)[:, :_C]
    quant_out = rows.reshape(_B, _HW, _C).transpose(0, 2, 1).reshape(
        _B, _C, _H, _W)
    loss_s = loss[0, 0] * (1.2 / (_B * _C * _HW))
    encoding_indices = idx.reshape(_B, _H, _W)
    return quant_out, loss_s, encoding_indices


# speedup vs baseline: 2.9046x; 2.9046x over previous
import jax
import jax.numpy as jnp
from jax.experimental import pallas as pl
from jax.experimental.pallas import tpu as pltpu

_B, _C, _H, _W, _K = 8, 64, 64, 64, 1024
_HW = _H * _W
_T = 4096
_NB = _HW // _T


def _vq_block(x_ref, w_ref, out_ref, idx_ref, loss_ref, a_ref, b2_ref,
              w2_ref):
    b = pl.program_id(0)
    nb = pl.program_id(1)

    @pl.when(jnp.logical_and(b == 0, nb == 0))
    def _build_consts():
        w = w_ref[...]
        ones = jnp.ones((_K, 1), jnp.float32)
        kio = jax.lax.broadcasted_iota(jnp.int32, (_K, 1), 0)
        khi = (kio // 128).astype(jnp.float32)
        klo = (kio % 128).astype(jnp.float32)
        a = jnp.concatenate([w, ones, khi, klo], axis=1)
        a_ref[...] = a.astype(jnp.bfloat16)
        b2_ref[...] = jnp.sum(w * w, axis=1, keepdims=True)
        w2_ref[...] = w + w
        loss_ref[...] = jnp.zeros_like(loss_ref)

    x = x_ref[0].reshape(_C, _T)
    s2 = jnp.dot(w2_ref[...], x,
                 preferred_element_type=jnp.float32)
    a2 = jnp.sum(x * x, axis=0, keepdims=True)
    m = (a2 + b2_ref[...]) - s2
    minm = jnp.min(m, axis=0, keepdims=True)
    mask = m == minm
    oh = mask.astype(jnp.bfloat16)
    dn = (((0,), (0,)), ((), ()))
    g67 = jax.lax.dot_general(a_ref[...], oh, dn,
                              preferred_element_type=jnp.float32)
    cnt = g67[_C:_C + 1, :]
    out_ref[0] = g67[:_C, :].reshape(_C, _H, _W)
    idx_ref[0] = (g67[_C + 1:_C + 2, :] * 128.0
                  + g67[_C + 2:_C + 3, :]).astype(jnp.int32).reshape(_H, _W)

    @pl.when(jnp.sum(cnt) > _T + 0.5)
    def _resolve_ties():
        kiota = jax.lax.broadcasted_iota(jnp.int32, m.shape, 0).astype(
            jnp.float32)
        idxf = jnp.min(jnp.where(mask, kiota, float(_K)), axis=0,
                       keepdims=True)
        oh1 = (kiota == idxf).astype(jnp.bfloat16)
        g1 = jax.lax.dot_general(a_ref[...], oh1, dn,
                                 preferred_element_type=jnp.float32)
        out_ref[0] = g1[:_C, :].reshape(_C, _H, _W)
        idx_ref[0] = idxf.astype(jnp.int32).reshape(_H, _W)

    loss_ref[...] += jnp.sum(minm, axis=(0, 1), keepdims=True) * (
        1.2 / (_B * _C * _HW))


def kernel(quant_input, weight):
    quant_out, idx, loss = pl.pallas_call(
        _vq_block,
        grid=(_B, _NB),
        in_specs=[
            pl.BlockSpec((1, _C, _H, _W), lambda b, n: (b, 0, n, 0)),
            pl.BlockSpec((_K, _C), lambda b, n: (0, 0)),
        ],
        out_specs=[
            pl.BlockSpec((1, _C, _H, _W), lambda b, n: (b, 0, n, 0)),
            pl.BlockSpec((1, _H, _W), lambda b, n: (b, n, 0)),
            pl.BlockSpec((1, 1), lambda b, n: (0, 0)),
        ],
        out_shape=[
            jax.ShapeDtypeStruct((_B, _C, _H, _W), jnp.float32),
            jax.ShapeDtypeStruct((_B, _H, _W), jnp.int32),
            jax.ShapeDtypeStruct((1, 1), jnp.float32),
        ],
        scratch_shapes=[
            pltpu.VMEM((_K, _C + 3), jnp.bfloat16),
            pltpu.VMEM((_K, 1), jnp.float32),
            pltpu.VMEM((_K, _C), jnp.float32),
        ],
    )(quant_input, weight)
    loss_s = loss[0, 0]
    return quant_out, loss_s, idx
